# Initial kernel scaffold; baseline (speedup 1.0000x reference)
import jax
import jax.numpy as jnp
from jax.experimental import pallas as pl


def kernel(x):
    raise NotImplementedError("write your pallas kernel here")


# speedup vs baseline: 2.2477x; 2.2477x over previous
import functools

import jax
import jax.numpy as jnp
from jax import lax
from jax.experimental import pallas as pl
from jax.experimental.pallas import tpu as pltpu
from jax.experimental.pallas import tpu_sc as plsc

ROWS, COLS = 4096, 8192
LANES = 16
NUM_CORES, NUM_SUBCORES = 2, 16
NW = NUM_CORES * NUM_SUBCORES
ROWS_PER_W = ROWS // NW
R = 4
NBLK = ROWS_PER_W // R
NCHUNK = COLS // LANES


def _scan_body(x_hbm, out_hbm, buf):
    c = lax.axis_index("c")
    s = lax.axis_index("s")
    wid = s * NUM_CORES + c
    base_row = wid * ROWS_PER_W

    def block(b, carry_unused):
        row0 = base_row + b * R
        pltpu.sync_copy(x_hbm.at[pl.ds(row0, R)], buf)

        def chunk(j, carries):
            col = j * LANES
            new = []
            for r in range(R):
                v = buf[r, pl.ds(col, LANES)]
                inc = plsc.cumsum(v)
                buf[r, pl.ds(col, LANES)] = inc - v + carries[r]
                new.append(carries[r] + jnp.sum(v))
            return tuple(new)

        lax.fori_loop(0, NCHUNK, chunk,
                      tuple(jnp.float32(0.0) for _ in range(R)))
        pltpu.sync_copy(buf, out_hbm.at[pl.ds(row0, R)])
        return carry_unused

    lax.fori_loop(0, NBLK, block, 0)


@jax.jit
def kernel(x):
    mesh = plsc.VectorSubcoreMesh(
        core_axis_name="c", subcore_axis_name="s",
        num_cores=NUM_CORES, num_subcores=NUM_SUBCORES)
    f = pl.kernel(
        _scan_body,
        out_type=jax.ShapeDtypeStruct((ROWS, COLS), jnp.float32),
        mesh=mesh,
        scratch_types=[pltpu.VMEM((R, COLS), jnp.float32)],
        compiler_params=pltpu.CompilerParams(needs_layout_passes=False),
    )
    return f(x)
